# Initial kernel scaffold; baseline (speedup 1.0000x reference)
import functools

import jax
import jax.numpy as jnp
from jax import lax
from jax.experimental import pallas as pl
from jax.experimental.pallas import tpu as pltpu
from jax.experimental.pallas import tpu_sc as plsc

N_NODES = 10000
N_PAD = 10240
N_EDGES = 320000
CHUNK = 80
N_CHUNKS = N_EDGES // CHUNK


def _make_hist_kernel(nc, ns):
    nw = nc * ns
    chunks_per_tile = N_CHUNKS // nw
    rows_per_tile = N_PAD // nw

    @functools.partial(
        pl.kernel,
        mesh=plsc.VectorSubcoreMesh(core_axis_name="c", subcore_axis_name="s"),
        out_type=jax.ShapeDtypeStruct((nc * N_PAD,), jnp.float32),
        scratch_types=[
            pltpu.VMEM((chunks_per_tile, CHUNK), jnp.int32),
            pltpu.VMEM((CHUNK,), jnp.float32),
            pltpu.VMEM_SHARED((N_PAD,), jnp.float32),
        ],
    )
    def hist_kernel(dst_hbm, out_hbm, dst_v, ones_v, hist_s):
        c = lax.axis_index("c")
        s = lax.axis_index("s")
        w = s * nc + c
        pltpu.sync_copy(
            dst_hbm.at[pl.ds(w * chunks_per_tile, chunks_per_tile)], dst_v)
        for j in range(CHUNK // 16):
            ones_v[pl.ds(j * 16, 16)] = jnp.zeros((16,), jnp.float32)
        base = s * rows_per_tile
        for j in range(rows_per_tile // CHUNK):
            pltpu.sync_copy(ones_v, hist_s.at[pl.ds(base + j * CHUNK, CHUNK)])
        for j in range(CHUNK // 16):
            ones_v[pl.ds(j * 16, 16)] = jnp.ones((16,), jnp.float32)
        plsc.subcore_barrier()

        def body(i, carry):
            pltpu.sync_copy(ones_v, hist_s.at[dst_v.at[i]], add=True)
            return carry

        lax.fori_loop(0, chunks_per_tile, body, 0)
        plsc.subcore_barrier()
        pltpu.sync_copy(
            hist_s.at[pl.ds(base, rows_per_tile)],
            out_hbm.at[pl.ds(c * N_PAD + base, rows_per_tile)])

    return hist_kernel


def _make_agg_kernel(nc, ns, d):
    nw = nc * ns
    chunks_per_tile = N_CHUNKS // nw
    rows_per_tile = N_PAD // nw

    @functools.partial(
        pl.kernel,
        mesh=plsc.VectorSubcoreMesh(core_axis_name="c", subcore_axis_name="s"),
        out_type=jax.ShapeDtypeStruct((nc * N_PAD, d), jnp.float32),
        scratch_types=[
            pltpu.VMEM((chunks_per_tile, CHUNK), jnp.int32),
            pltpu.VMEM((chunks_per_tile, CHUNK), jnp.int32),
            pltpu.VMEM((CHUNK, d), jnp.float32),
            pltpu.VMEM_SHARED((N_PAD, d), jnp.float32),
            pltpu.SemaphoreType.DMA,
        ],
    )
    def agg_kernel(g_hbm, src_hbm, dst_hbm, out_hbm,
                   src_v, dst_v, buf, acc_s, sem):
        c = lax.axis_index("c")
        s = lax.axis_index("s")
        w = s * nc + c
        pltpu.sync_copy(
            src_hbm.at[pl.ds(w * chunks_per_tile, chunks_per_tile)], src_v)
        pltpu.sync_copy(
            dst_hbm.at[pl.ds(w * chunks_per_tile, chunks_per_tile)], dst_v)

        def zbody(i, carry):
            for k2 in range(d // 16):
                buf[i, pl.ds(k2 * 16, 16)] = jnp.zeros((16,), jnp.float32)
            return carry

        lax.fori_loop(0, CHUNK, zbody, 0)
        base = s * rows_per_tile
        for j in range(rows_per_tile // CHUNK):
            pltpu.sync_copy(buf, acc_s.at[pl.ds(base + j * CHUNK, CHUNK)])
        plsc.subcore_barrier()

        def body(i, carry):
            pltpu.async_copy(g_hbm.at[src_v.at[i]], buf, sem).wait()
            pltpu.sync_copy(buf, acc_s.at[dst_v.at[i]], add=True)
            return carry

        lax.fori_loop(0, chunks_per_tile, body, 0)
        plsc.subcore_barrier()
        pltpu.sync_copy(
            acc_s.at[pl.ds(base, rows_per_tile)],
            out_hbm.at[pl.ds(c * N_PAD + base, rows_per_tile)])

    return agg_kernel


_BLK = 512


def _dense1(hist, x_pad, W1):
    nc = hist.shape[0]

    def body(hist_ref, x_ref, w_ref, o_ref):
        deg = jnp.sum(hist_ref[...], axis=0) + 1.0
        dis = lax.rsqrt(deg)
        h = jnp.dot(x_ref[...], w_ref[...], preferred_element_type=jnp.float32)
        o_ref[...] = h * dis[:, None]

    return pl.pallas_call(
        body,
        grid=(N_PAD // _BLK,),
        in_specs=[
            pl.BlockSpec((nc, _BLK), lambda i: (0, i)),
            pl.BlockSpec((_BLK, 128), lambda i: (i, 0)),
            pl.BlockSpec((128, 128), lambda i: (0, 0)),
        ],
        out_specs=pl.BlockSpec((_BLK, 128), lambda i: (i, 0)),
        out_shape=jax.ShapeDtypeStruct((N_PAD, 128), jnp.float32),
    )(hist, x_pad, W1)


def _dense2(hist, agg1, g1, W2, b1):
    nc = hist.shape[0]

    def body(hist_ref, agg_ref, g1_ref, w_ref, b_ref, o_ref):
        deg = jnp.sum(hist_ref[...], axis=0) + 1.0
        dis = lax.rsqrt(deg)
        aggsum = jnp.sum(agg_ref[...], axis=0)
        h = dis[:, None] * (aggsum + g1_ref[...]) + b_ref[...]
        h = jnp.maximum(h, 0.0)
        o_ref[...] = jnp.dot(
            h, w_ref[...], preferred_element_type=jnp.float32) * dis[:, None]

    return pl.pallas_call(
        body,
        grid=(N_PAD // _BLK,),
        in_specs=[
            pl.BlockSpec((nc, _BLK), lambda i: (0, i)),
            pl.BlockSpec((nc, _BLK, 128), lambda i: (0, i, 0)),
            pl.BlockSpec((_BLK, 128), lambda i: (i, 0)),
            pl.BlockSpec((128, 64), lambda i: (0, 0)),
            pl.BlockSpec((1, 128), lambda i: (0, 0)),
        ],
        out_specs=pl.BlockSpec((_BLK, 64), lambda i: (i, 0)),
        out_shape=jax.ShapeDtypeStruct((N_PAD, 64), jnp.float32),
    )(hist, agg1, g1, W2, b1)


def _final(hist, agg2, g2, b2):
    nc = hist.shape[0]

    def body(hist_ref, agg_ref, g2_ref, b_ref, o_ref):
        deg = jnp.sum(hist_ref[...], axis=0) + 1.0
        dis = lax.rsqrt(deg)
        z = dis[:, None] * (jnp.sum(agg_ref[...], axis=0) + g2_ref[...]) \
            + b_ref[...]
        m = jnp.max(z, axis=1, keepdims=True)
        e = jnp.exp(z - m)
        lse = jnp.log(jnp.sum(e, axis=1, keepdims=True)) + m
        o_ref[...] = z - lse

    return pl.pallas_call(
        body,
        grid=(N_PAD // _BLK,),
        in_specs=[
            pl.BlockSpec((nc, _BLK), lambda i: (0, i)),
            pl.BlockSpec((nc, _BLK, 64), lambda i: (0, i, 0)),
            pl.BlockSpec((_BLK, 64), lambda i: (i, 0)),
            pl.BlockSpec((1, 64), lambda i: (0, 0)),
        ],
        out_specs=pl.BlockSpec((_BLK, 64), lambda i: (i, 0)),
        out_shape=jax.ShapeDtypeStruct((N_PAD, 64), jnp.float32),
    )(hist, agg2, g2, b2)


def kernel(x, edge_index, W1, b1, W2, b2):
    info = plsc.get_sparse_core_info()
    nc, ns = info.num_cores, info.num_subcores

    ei = edge_index.astype(jnp.int32)
    src_mat = ei[0].reshape(N_CHUNKS, CHUNK)
    dst_mat = ei[1].reshape(N_CHUNKS, CHUNK)

    hist = _make_hist_kernel(nc, ns)(dst_mat).reshape(nc, N_PAD)
    x_pad = jnp.pad(x, ((0, N_PAD - N_NODES), (0, 0)))
    g1 = _dense1(hist, x_pad, W1)
    agg1 = _make_agg_kernel(nc, ns, 128)(g1, src_mat, dst_mat)
    agg1 = agg1.reshape(nc, N_PAD, 128)
    g2 = _dense2(hist, agg1, g1, W2, b1.reshape(1, 128))
    agg2 = _make_agg_kernel(nc, ns, 64)(g2, src_mat, dst_mat)
    agg2 = agg2.reshape(nc, N_PAD, 64)
    out = _final(hist, agg2, g2, b2.reshape(1, 64))
    return out[:N_NODES]


# speedup vs baseline: 22.6546x; 22.6546x over previous
import functools

import jax
import jax.numpy as jnp
from jax import lax
from jax.experimental import pallas as pl
from jax.experimental.pallas import tpu as pltpu
from jax.experimental.pallas import tpu_sc as plsc

N_NODES = 10000
N_PAD = 10240
N_EDGES = 320000
CHUNK = 125
N_CHUNKS = N_EDGES // CHUNK
ZBLK = 64


def _make_hist_kernel(nc, ns):
    nw = nc * ns
    chunks_per_tile = N_CHUNKS // nw
    rows_per_tile = N_PAD // nw

    @functools.partial(
        pl.kernel,
        mesh=plsc.VectorSubcoreMesh(core_axis_name="c", subcore_axis_name="s"),
        out_type=jax.ShapeDtypeStruct((nc * N_PAD,), jnp.float32),
        scratch_types=[
            pltpu.VMEM((chunks_per_tile, CHUNK), jnp.int32),
            pltpu.VMEM((128,), jnp.float32),
            pltpu.VMEM((rows_per_tile,), jnp.float32),
            pltpu.VMEM_SHARED((N_PAD,), jnp.float32),
        ],
    )
    def hist_kernel(dst_hbm, out_hbm, dst_v, val_v, stage_v, hist_s):
        c = lax.axis_index("c")
        s = lax.axis_index("s")
        w = s * nc + c
        pltpu.sync_copy(
            dst_hbm.at[pl.ds(w * chunks_per_tile, chunks_per_tile)], dst_v)
        for j in range(128 // 16):
            val_v[pl.ds(j * 16, 16)] = jnp.zeros((16,), jnp.float32)
        base = s * rows_per_tile
        for j in range(rows_per_tile // ZBLK):
            pltpu.sync_copy(val_v.at[pl.ds(0, ZBLK)],
                            hist_s.at[pl.ds(base + j * ZBLK, ZBLK)])
        for j in range(128 // 16):
            val_v[pl.ds(j * 16, 16)] = jnp.ones((16,), jnp.float32)
        plsc.subcore_barrier()

        def body(i, carry):
            pltpu.sync_copy(val_v.at[pl.ds(0, CHUNK)],
                            hist_s.at[dst_v.at[i]], add=True)
            return carry

        lax.fori_loop(0, chunks_per_tile, body, 0)
        plsc.subcore_barrier()
        pltpu.sync_copy(hist_s.at[pl.ds(base, rows_per_tile)], stage_v)
        pltpu.sync_copy(
            stage_v, out_hbm.at[pl.ds(c * N_PAD + base, rows_per_tile)])

    return hist_kernel


def _make_agg_kernel(nc, ns, d):
    nw = nc * ns
    chunks_per_tile = N_CHUNKS // nw
    rows_per_tile = N_PAD // nw

    @functools.partial(
        pl.kernel,
        mesh=plsc.VectorSubcoreMesh(core_axis_name="c", subcore_axis_name="s"),
        out_type=jax.ShapeDtypeStruct((nc * N_PAD, d), jnp.float32),
        scratch_types=[
            pltpu.VMEM((chunks_per_tile, CHUNK), jnp.int32),
            pltpu.VMEM((chunks_per_tile, CHUNK), jnp.int32),
            pltpu.VMEM((CHUNK, d), jnp.float32),
            pltpu.VMEM((ZBLK, d), jnp.float32),
            pltpu.VMEM_SHARED((N_PAD, d), jnp.float32),
            pltpu.SemaphoreType.DMA,
        ],
    )
    def agg_kernel(g_hbm, src_hbm, dst_hbm, out_hbm,
                   src_v, dst_v, buf, zbuf, acc_s, sem):
        c = lax.axis_index("c")
        s = lax.axis_index("s")
        w = s * nc + c
        pltpu.sync_copy(
            src_hbm.at[pl.ds(w * chunks_per_tile, chunks_per_tile)], src_v)
        pltpu.sync_copy(
            dst_hbm.at[pl.ds(w * chunks_per_tile, chunks_per_tile)], dst_v)

        def zbody(i, carry):
            for k2 in range(d // 16):
                zbuf[i, pl.ds(k2 * 16, 16)] = jnp.zeros((16,), jnp.float32)
            return carry

        lax.fori_loop(0, ZBLK, zbody, 0)
        base = s * rows_per_tile
        for j in range(rows_per_tile // ZBLK):
            pltpu.sync_copy(zbuf, acc_s.at[pl.ds(base + j * ZBLK, ZBLK)])
        plsc.subcore_barrier()

        def body(i, carry):
            pltpu.async_copy(g_hbm.at[src_v.at[i]], buf, sem).wait()
            pltpu.sync_copy(buf, acc_s.at[dst_v.at[i]], add=True)
            return carry

        lax.fori_loop(0, chunks_per_tile, body, 0)
        plsc.subcore_barrier()
        for j in range(rows_per_tile // ZBLK):
            pltpu.sync_copy(acc_s.at[pl.ds(base + j * ZBLK, ZBLK)], zbuf)
            pltpu.sync_copy(
                zbuf, out_hbm.at[pl.ds(c * N_PAD + base + j * ZBLK, ZBLK)])

    return agg_kernel


_BLK = 512


def _dense1(hist, x_pad, W1):
    nc = hist.shape[0]

    def body(hist_ref, x_ref, w_ref, o_ref):
        deg = jnp.sum(hist_ref[...], axis=0) + 1.0
        dis = lax.rsqrt(deg)
        h = jnp.dot(x_ref[...], w_ref[...], preferred_element_type=jnp.float32)
        o_ref[...] = h * dis[:, None]

    return pl.pallas_call(
        body,
        grid=(N_PAD // _BLK,),
        in_specs=[
            pl.BlockSpec((nc, _BLK), lambda i: (0, i)),
            pl.BlockSpec((_BLK, 128), lambda i: (i, 0)),
            pl.BlockSpec((128, 128), lambda i: (0, 0)),
        ],
        out_specs=pl.BlockSpec((_BLK, 128), lambda i: (i, 0)),
        out_shape=jax.ShapeDtypeStruct((N_PAD, 128), jnp.float32),
    )(hist, x_pad, W1)


def _dense2(hist, agg1, g1, W2, b1):
    nc = hist.shape[0]

    def body(hist_ref, agg_ref, g1_ref, w_ref, b_ref, o_ref):
        deg = jnp.sum(hist_ref[...], axis=0) + 1.0
        dis = lax.rsqrt(deg)
        aggsum = jnp.sum(agg_ref[...], axis=0)
        h = dis[:, None] * (aggsum + g1_ref[...]) + b_ref[...]
        h = jnp.maximum(h, 0.0)
        o_ref[...] = jnp.dot(
            h, w_ref[...], preferred_element_type=jnp.float32) * dis[:, None]

    return pl.pallas_call(
        body,
        grid=(N_PAD // _BLK,),
        in_specs=[
            pl.BlockSpec((nc, _BLK), lambda i: (0, i)),
            pl.BlockSpec((nc, _BLK, 128), lambda i: (0, i, 0)),
            pl.BlockSpec((_BLK, 128), lambda i: (i, 0)),
            pl.BlockSpec((128, 128), lambda i: (0, 0)),
            pl.BlockSpec((1, 128), lambda i: (0, 0)),
        ],
        out_specs=pl.BlockSpec((_BLK, 128), lambda i: (i, 0)),
        out_shape=jax.ShapeDtypeStruct((N_PAD, 128), jnp.float32),
    )(hist, agg1, g1, W2, b1)


def _final(hist, agg2, g2, b2):
    nc = hist.shape[0]

    def body(hist_ref, agg_ref, g2_ref, b_ref, o_ref):
        deg = jnp.sum(hist_ref[...], axis=0) + 1.0
        dis = lax.rsqrt(deg)
        zfull = jnp.sum(agg_ref[...], axis=0) + g2_ref[...]
        z = dis[:, None] * zfull[:, :64] + b_ref[...]
        m = jnp.max(z, axis=1, keepdims=True)
        e = jnp.exp(z - m)
        lse = jnp.log(jnp.sum(e, axis=1, keepdims=True)) + m
        o_ref[...] = z - lse

    return pl.pallas_call(
        body,
        grid=(N_PAD // _BLK,),
        in_specs=[
            pl.BlockSpec((nc, _BLK), lambda i: (0, i)),
            pl.BlockSpec((nc, _BLK, 128), lambda i: (0, i, 0)),
            pl.BlockSpec((_BLK, 128), lambda i: (i, 0)),
            pl.BlockSpec((1, 64), lambda i: (0, 0)),
        ],
        out_specs=pl.BlockSpec((_BLK, 64), lambda i: (i, 0)),
        out_shape=jax.ShapeDtypeStruct((N_PAD, 64), jnp.float32),
    )(hist, agg2, g2, b2)


def kernel(x, edge_index, W1, b1, W2, b2):
    info = plsc.get_sparse_core_info()
    nc, ns = info.num_cores, info.num_subcores

    ei = edge_index.astype(jnp.int32)
    src_mat = ei[0].reshape(N_CHUNKS, CHUNK)
    dst_mat = ei[1].reshape(N_CHUNKS, CHUNK)

    hist = _make_hist_kernel(nc, ns)(dst_mat).reshape(nc, N_PAD)
    x_pad = jnp.pad(x, ((0, N_PAD - N_NODES), (0, 0)))
    g1 = _dense1(hist, x_pad, W1)
    agg1 = _make_agg_kernel(nc, ns, 128)(g1, src_mat, dst_mat)
    agg1 = agg1.reshape(nc, N_PAD, 128)
    W2p = jnp.pad(W2, ((0, 0), (0, 128 - W2.shape[1])))
    g2 = _dense2(hist, agg1, g1, W2p, b1.reshape(1, 128))
    agg2 = _make_agg_kernel(nc, ns, 128)(g2, src_mat, dst_mat)
    agg2 = agg2.reshape(nc, N_PAD, 128)
    out = _final(hist, agg2, g2, b2.reshape(1, 64))
    return out[:N_NODES]
